# baseline (speedup 1.0000x reference)
import jax
import jax.numpy as jnp
from jax.experimental import pallas as pl

BS = 256


def _body(x_ref, t_ref, o_ref):
    o_ref[...] = x_ref[...] + t_ref[...][None, :, :]


def kernel(input_embeddings, emb_table):
    B, S, D = input_embeddings.shape
    ns = S // BS
    return pl.pallas_call(
        _body,
        grid=(ns, B),
        in_specs=[
            pl.BlockSpec((1, BS, D), lambda s, b: (b, s, 0)),
            pl.BlockSpec((BS, D), lambda s, b: (s, 0)),
        ],
        out_specs=pl.BlockSpec((1, BS, D), lambda s, b: (b, s, 0)),
        out_shape=jax.ShapeDtypeStruct((B, S, D), input_embeddings.dtype),
    )(input_embeddings, emb_table[:S])


# speedup vs baseline: 1.4748x; 1.4748x over previous
import jax
import jax.numpy as jnp
from jax.experimental import pallas as pl

BS = 1024


def _body(x_ref, t_ref, o_ref):
    o_ref[...] = x_ref[...] + t_ref[...][None, :, :]


def kernel(input_embeddings, emb_table):
    B, S, D = input_embeddings.shape
    ns = S // BS
    return pl.pallas_call(
        _body,
        grid=(ns, B),
        in_specs=[
            pl.BlockSpec((1, BS, D), lambda s, b: (b, s, 0)),
            pl.BlockSpec((BS, D), lambda s, b: (s, 0)),
        ],
        out_specs=pl.BlockSpec((1, BS, D), lambda s, b: (b, s, 0)),
        out_shape=jax.ShapeDtypeStruct((B, S, D), input_embeddings.dtype),
    )(input_embeddings, emb_table[:S])


# speedup vs baseline: 1.5368x; 1.0421x over previous
import jax
import jax.numpy as jnp
from jax.experimental import pallas as pl

BS = 2048


def _body(x_ref, t_ref, o_ref):
    o_ref[...] = x_ref[...] + t_ref[...][None, :, :]


def kernel(input_embeddings, emb_table):
    B, S, D = input_embeddings.shape
    ns = S // BS
    return pl.pallas_call(
        _body,
        grid=(ns, B),
        in_specs=[
            pl.BlockSpec((1, BS, D), lambda s, b: (b, s, 0)),
            pl.BlockSpec((BS, D), lambda s, b: (s, 0)),
        ],
        out_specs=pl.BlockSpec((1, BS, D), lambda s, b: (b, s, 0)),
        out_shape=jax.ShapeDtypeStruct((B, S, D), input_embeddings.dtype),
    )(input_embeddings, emb_table[:S])
